# Initial kernel scaffold; baseline (speedup 1.0000x reference)
import jax
import jax.numpy as jnp
from jax.experimental import pallas as pl


def kernel(hn, he, fe, fes, norm, edge_index, We1, be1, We2, Wf1, Wf2, Wu1, bu1, Wu2, Wn1, bn1, Wn2):
    raise NotImplementedError("write your pallas kernel here")


# speedup vs baseline: 1.8083x; 1.8083x over previous
import functools
import math

import jax
import jax.numpy as jnp
import numpy as np
from jax import lax
from jax.experimental import pallas as pl
from jax.experimental.pallas import tpu as pltpu
from jax.experimental.pallas import tpu_sc as plsc

NC = 2
NS = 16
NW = NC * NS
CH = 80

def _mk_mesh():
    return plsc.VectorSubcoreMesh(core_axis_name="c", subcore_axis_name="s",
                                  num_cores=NC, num_subcores=NS)


def _gather_call(hn_f32, src, dst):
    n, d = hn_f32.shape
    e = src.shape[0]
    ew = e // NW
    nch = ew // CH

    @functools.partial(
        pl.kernel,
        mesh=_mk_mesh(),
        out_type=(
            jax.ShapeDtypeStruct((e, d), jnp.float32),
            jax.ShapeDtypeStruct((e, d), jnp.float32),
        ),
        scratch_types=[
            pltpu.VMEM((CH,), jnp.int32),
            pltpu.VMEM((CH,), jnp.int32),
            pltpu.VMEM((CH, d), jnp.float32),
            pltpu.VMEM((CH, d), jnp.float32),
            pltpu.SemaphoreType.DMA,
            pltpu.SemaphoreType.DMA,
        ],
    )
    def k(hn_hbm, src_hbm, dst_hbm, hs_hbm, hd_hbm, sidx, didx, srows, drows,
          sem1, sem2):
        wid = lax.axis_index("s") * NC + lax.axis_index("c")

        def body(j, carry):
            base = wid * ew + j * CH
            pltpu.sync_copy(src_hbm.at[pl.ds(base, CH)], sidx)
            pltpu.sync_copy(dst_hbm.at[pl.ds(base, CH)], didx)
            c1 = pltpu.async_copy(hn_hbm.at[sidx], srows, sem1)
            c2 = pltpu.async_copy(hn_hbm.at[didx], drows, sem2)
            c1.wait()
            c2.wait()
            pltpu.sync_copy(srows, hs_hbm.at[pl.ds(base, CH)])
            pltpu.sync_copy(drows, hd_hbm.at[pl.ds(base, CH)])
            return carry

        lax.fori_loop(0, nch, body, 0)

    return k(hn_f32, src, dst)


def _scatter_call(scaled, dst, zrows):
    e, d = scaled.shape
    np_ = zrows.shape[0]
    ew = e // NW
    nch = ew // CH
    npt = np_ // NS

    @functools.partial(
        pl.kernel,
        mesh=_mk_mesh(),
        out_type=jax.ShapeDtypeStruct((NC, np_, d), jnp.float32),
        scratch_types=[
            pltpu.VMEM((CH,), jnp.int32),
            pltpu.VMEM((CH, d), jnp.float32),
            pltpu.VMEM_SHARED((np_, d), jnp.float32),
        ],
    )
    def k(scaled_hbm, dst_hbm, z_hbm, out_hbm, idx_v, rows_v, shared):
        cid = lax.axis_index("c")
        sid = lax.axis_index("s")
        wid = sid * NC + cid
        pltpu.sync_copy(z_hbm.at[pl.ds(sid * npt, npt)],
                        shared.at[pl.ds(sid * npt, npt)])
        plsc.subcore_barrier()

        def body(j, carry):
            base = wid * ew + j * CH
            pltpu.sync_copy(dst_hbm.at[pl.ds(base, CH)], idx_v)
            pltpu.sync_copy(scaled_hbm.at[pl.ds(base, CH)], rows_v)
            pltpu.sync_copy(rows_v, shared.at[idx_v], add=True)
            return carry

        lax.fori_loop(0, nch, body, 0)
        plsc.subcore_barrier()
        pltpu.sync_copy(shared.at[pl.ds(sid * npt, npt)],
                        out_hbm.at[cid, pl.ds(sid * npt, npt)])

    return k(scaled, dst, zrows)


def _edge_body(he_ref, hs_ref, hd_ref, fe_ref, fes_ref, norm_ref,
               we1_ref, be1_ref, we2_ref, wf1_ref, wf2_ref,
               wu1_ref, bu1_ref, wu2_ref, he_new_ref, scaled_ref):
    f32 = jnp.float32
    he = he_ref[...]
    hs = hs_ref[...].astype(jnp.bfloat16)
    hd = hd_ref[...].astype(jnp.bfloat16)
    d = he.shape[1]
    x1 = jnp.dot(he.astype(jnp.bfloat16), we1_ref[0:d, :],
                 preferred_element_type=f32)
    x1 += jnp.dot(hs, we1_ref[d:2 * d, :], preferred_element_type=f32)
    x1 += jnp.dot(hd, we1_ref[2 * d:3 * d, :], preferred_element_type=f32)
    x1 += be1_ref[...]
    h1 = x1 * (1.0 / (1.0 + jnp.exp(-x1)))
    v = jnp.dot(h1, we2_ref[...], preferred_element_type=f32)
    eb = fes_ref.shape[1]
    fch = wf1_ref.shape[1]
    r = jnp.maximum(
        jnp.dot(fes_ref[...], wf1_ref[...], preferred_element_type=f32), 0.0)
    w3 = jnp.dot(r, wf2_ref[...], preferred_element_type=f32) * (
        1.0 / math.sqrt(eb) / math.sqrt(fch))
    p = v * fe_ref[...]
    d0 = jnp.sum(p[:, 0:1], axis=1, keepdims=True)
    d1 = jnp.sum(p[:, 1:4], axis=1, keepdims=True) * (1.0 / math.sqrt(3.0))
    d2 = jnp.sum(p[:, 4:9], axis=1, keepdims=True) * (1.0 / math.sqrt(5.0))
    tp = (w3[:, 0:d] * d0 + w3[:, d:2 * d] * d1 + w3[:, 2 * d:3 * d] * d2) * (
        1.0 / math.sqrt(3.0))
    u1 = jnp.dot(tp.astype(jnp.bfloat16), wu1_ref[0:d, :],
                 preferred_element_type=f32)
    u1 += jnp.dot(hs, wu1_ref[d:2 * d, :], preferred_element_type=f32)
    u1 += jnp.dot(hd, wu1_ref[2 * d:3 * d, :], preferred_element_type=f32)
    u1 += bu1_ref[...]
    h2 = u1 * (1.0 / (1.0 + jnp.exp(-u1)))
    he_new = he + jnp.dot(h2.astype(jnp.bfloat16), wu2_ref[...],
                          preferred_element_type=f32)
    he_new_ref[...] = he_new
    scaled_ref[...] = he_new * norm_ref[...]


def _edge_call(he, hs, hd, fe, fes, normc, we1, be1, we2, wf1, wf2,
               wu1, bu1, wu2, be):
    e, d = he.shape
    hid = we1.shape[1]
    dsh = fe.shape[1]
    eb = fes.shape[1]
    fch = wf1.shape[1]
    grid = (e // be,)
    row = lambda i: (i, 0)
    full = lambda i: (0, 0)
    return pl.pallas_call(
        _edge_body,
        grid=grid,
        in_specs=[
            pl.BlockSpec((be, d), row),
            pl.BlockSpec((be, d), row),
            pl.BlockSpec((be, d), row),
            pl.BlockSpec((be, dsh), row),
            pl.BlockSpec((be, eb), row),
            pl.BlockSpec((be, 1), row),
            pl.BlockSpec((3 * d, hid), full),
            pl.BlockSpec((1, hid), full),
            pl.BlockSpec((hid, dsh), full),
            pl.BlockSpec((eb, fch), full),
            pl.BlockSpec((fch, 3 * d), full),
            pl.BlockSpec((3 * d, hid), full),
            pl.BlockSpec((1, hid), full),
            pl.BlockSpec((hid, d), full),
        ],
        out_specs=[
            pl.BlockSpec((be, d), row),
            pl.BlockSpec((be, d), row),
        ],
        out_shape=[
            jax.ShapeDtypeStruct((e, d), jnp.float32),
            jax.ShapeDtypeStruct((e, d), jnp.float32),
        ],
        compiler_params=pltpu.CompilerParams(
            dimension_semantics=("arbitrary",)),
    )(he, hs, hd, fe, fes, normc, we1, be1, we2, wf1, wf2, wu1, bu1, wu2)


def _node_body(hn_ref, p0_ref, p1_ref, wn1_ref, bn1_ref, wn2_ref, out_ref):
    f32 = jnp.float32
    hn = hn_ref[...]
    d = hn.shape[1]
    nf = p0_ref[...] + p1_ref[...]
    a = jnp.dot(hn.astype(jnp.bfloat16), wn1_ref[0:d, :],
                preferred_element_type=f32)
    a += jnp.dot(nf.astype(jnp.bfloat16), wn1_ref[d:2 * d, :],
                 preferred_element_type=f32)
    a += bn1_ref[...]
    h = a * (1.0 / (1.0 + jnp.exp(-a)))
    out_ref[...] = hn + jnp.dot(h.astype(jnp.bfloat16), wn2_ref[...],
                                preferred_element_type=f32)


def _node_call(hn, p0, p1, wn1, bn1, wn2, bn):
    n, d = hn.shape
    hid = wn1.shape[1]
    grid = (n // bn,)
    row = lambda i: (i, 0)
    full = lambda i: (0, 0)
    return pl.pallas_call(
        _node_body,
        grid=grid,
        in_specs=[
            pl.BlockSpec((bn, d), row),
            pl.BlockSpec((bn, d), row),
            pl.BlockSpec((bn, d), row),
            pl.BlockSpec((2 * d, hid), full),
            pl.BlockSpec((1, hid), full),
            pl.BlockSpec((hid, d), full),
        ],
        out_specs=pl.BlockSpec((bn, d), row),
        out_shape=jax.ShapeDtypeStruct((n, d), jnp.float32),
        compiler_params=pltpu.CompilerParams(
            dimension_semantics=("arbitrary",)),
    )(hn, p0, p1, wn1, bn1, wn2)


def kernel(hn, he, fe, fes, norm, edge_index, We1, be1, We2, Wf1, Wf2,
           Wu1, bu1, Wu2, Wn1, bn1, Wn2):
    n, d = hn.shape
    e = he.shape[0]
    bf = jnp.bfloat16
    src = edge_index[0]
    dst = edge_index[1]

    hs, hd = _gather_call(hn, src, dst)

    he_new, scaled = _edge_call(
        he, hs, hd, fe, fes, norm.reshape(e, 1),
        We1.astype(bf), be1.reshape(1, -1), We2, Wf1, Wf2,
        Wu1.astype(bf), bu1.reshape(1, -1), Wu2.astype(bf), be=1280)

    np_ = ((n + NS * 8 - 1) // (NS * 8)) * (NS * 8)
    partials = _scatter_call(scaled, dst, jnp.zeros((np_, d), jnp.float32))

    hn_new = _node_call(hn, partials[0, :n], partials[1, :n],
                        Wn1.astype(bf), bn1.reshape(1, -1), Wn2.astype(bf),
                        bn=2000)
    return hn_new, he_new


# speedup vs baseline: 2.0580x; 1.1381x over previous
import functools
import math

import jax
import jax.numpy as jnp
import numpy as np
from jax import lax
from jax.experimental import pallas as pl
from jax.experimental.pallas import tpu as pltpu
from jax.experimental.pallas import tpu_sc as plsc

NC = 2
NS = 16
NW = NC * NS
CH = 80

def _mk_mesh():
    return plsc.VectorSubcoreMesh(core_axis_name="c", subcore_axis_name="s",
                                  num_cores=NC, num_subcores=NS)


def _gather_call(hn_f32, src, dst):
    n, d = hn_f32.shape
    e = src.shape[0]
    ew = e // NW
    nch = ew // CH

    @functools.partial(
        pl.kernel,
        mesh=_mk_mesh(),
        out_type=(
            jax.ShapeDtypeStruct((e, d), jnp.float32),
            jax.ShapeDtypeStruct((e, d), jnp.float32),
        ),
        scratch_types=[
            pltpu.VMEM((CH,), jnp.int32),
            pltpu.VMEM((CH,), jnp.int32),
            pltpu.VMEM((CH, d), jnp.float32),
            pltpu.VMEM((CH, d), jnp.float32),
            pltpu.SemaphoreType.DMA,
            pltpu.SemaphoreType.DMA,
        ],
    )
    def k(hn_hbm, src_hbm, dst_hbm, hs_hbm, hd_hbm, sidx, didx, srows, drows,
          sem1, sem2):
        wid = lax.axis_index("s") * NC + lax.axis_index("c")

        def body(j, carry):
            base = wid * ew + j * CH
            pltpu.sync_copy(src_hbm.at[pl.ds(base, CH)], sidx)
            pltpu.sync_copy(dst_hbm.at[pl.ds(base, CH)], didx)
            c1 = pltpu.async_copy(hn_hbm.at[sidx], srows, sem1)
            c2 = pltpu.async_copy(hn_hbm.at[didx], drows, sem2)
            c1.wait()
            c2.wait()
            pltpu.sync_copy(srows, hs_hbm.at[pl.ds(base, CH)])
            pltpu.sync_copy(drows, hd_hbm.at[pl.ds(base, CH)])
            return carry

        lax.fori_loop(0, nch, body, 0)

    return k(hn_f32, src, dst)


def _scatter_call(scaled, dst, zrows):
    e, d = scaled.shape
    np_ = zrows.shape[0]
    ew = e // NW
    nch = ew // CH
    npt = np_ // NS

    @functools.partial(
        pl.kernel,
        mesh=_mk_mesh(),
        out_type=jax.ShapeDtypeStruct((NC, np_, d), jnp.float32),
        scratch_types=[
            pltpu.VMEM((CH,), jnp.int32),
            pltpu.VMEM((CH, d), jnp.float32),
            pltpu.VMEM_SHARED((np_, d), jnp.float32),
        ],
    )
    def k(scaled_hbm, dst_hbm, z_hbm, out_hbm, idx_v, rows_v, shared):
        cid = lax.axis_index("c")
        sid = lax.axis_index("s")
        wid = sid * NC + cid
        pltpu.sync_copy(z_hbm.at[pl.ds(sid * npt, npt)],
                        shared.at[pl.ds(sid * npt, npt)])
        plsc.subcore_barrier()

        def body(j, carry):
            base = wid * ew + j * CH
            pltpu.sync_copy(dst_hbm.at[pl.ds(base, CH)], idx_v)
            pltpu.sync_copy(scaled_hbm.at[pl.ds(base, CH)], rows_v)
            pltpu.sync_copy(rows_v, shared.at[idx_v], add=True)
            return carry

        lax.fori_loop(0, nch, body, 0)
        plsc.subcore_barrier()
        pltpu.sync_copy(shared.at[pl.ds(sid * npt, npt)],
                        out_hbm.at[cid, pl.ds(sid * npt, npt)])

    return k(scaled, dst, zrows)


_SUBTILES = 5


def _edge_body(he_ref, hs_ref, hd_ref, fe_ref, fes_ref, norm_ref,
               we1_ref, be1_ref, we2_ref, wf1_ref, wf2_ref,
               wu1_ref, bu1_ref, wu2_ref, he_new_ref, scaled_ref):
    f32 = jnp.float32
    bf = jnp.bfloat16
    be, d = he_ref.shape
    eb = fes_ref.shape[1]
    fch = wf1_ref.shape[1]
    st = be // _SUBTILES
    for t in range(_SUBTILES):
        rs = pl.ds(t * st, st)
        he = he_ref[rs, :]
        hs = hs_ref[rs, :].astype(bf)
        hd = hd_ref[rs, :].astype(bf)
        x1 = jnp.dot(he.astype(bf), we1_ref[0:d, :],
                     preferred_element_type=f32)
        x1 += jnp.dot(hs, we1_ref[d:2 * d, :], preferred_element_type=f32)
        x1 += jnp.dot(hd, we1_ref[2 * d:3 * d, :], preferred_element_type=f32)
        x1 += be1_ref[...]
        h1 = x1 * (0.5 * jnp.tanh(0.5 * x1) + 0.5)
        v = jnp.dot(h1, we2_ref[...], preferred_element_type=f32)
        r = jnp.maximum(
            jnp.dot(fes_ref[rs, :], wf1_ref[...],
                    preferred_element_type=f32), 0.0)
        w3 = jnp.dot(r, wf2_ref[...], preferred_element_type=f32) * (
            1.0 / math.sqrt(eb) / math.sqrt(fch))
        p = v * fe_ref[rs, :]
        d0 = jnp.sum(p[:, 0:1], axis=1, keepdims=True)
        d1 = jnp.sum(p[:, 1:4], axis=1, keepdims=True) * (
            1.0 / math.sqrt(3.0))
        d2 = jnp.sum(p[:, 4:9], axis=1, keepdims=True) * (
            1.0 / math.sqrt(5.0))
        tp = (w3[:, 0:d] * d0 + w3[:, d:2 * d] * d1 +
              w3[:, 2 * d:3 * d] * d2) * (1.0 / math.sqrt(3.0))
        u1 = jnp.dot(tp.astype(bf), wu1_ref[0:d, :],
                     preferred_element_type=f32)
        u1 += jnp.dot(hs, wu1_ref[d:2 * d, :], preferred_element_type=f32)
        u1 += jnp.dot(hd, wu1_ref[2 * d:3 * d, :], preferred_element_type=f32)
        u1 += bu1_ref[...]
        h2 = u1 * (0.5 * jnp.tanh(0.5 * u1) + 0.5)
        he_new = he + jnp.dot(h2.astype(bf), wu2_ref[...],
                              preferred_element_type=f32)
        he_new_ref[rs, :] = he_new
        scaled_ref[rs, :] = he_new * norm_ref[rs, :]


def _edge_call(he, hs, hd, fe, fes, normc, we1, be1, we2, wf1, wf2,
               wu1, bu1, wu2, be):
    e, d = he.shape
    hid = we1.shape[1]
    dsh = fe.shape[1]
    eb = fes.shape[1]
    fch = wf1.shape[1]
    grid = (e // be,)
    row = lambda i: (i, 0)
    full = lambda i: (0, 0)
    return pl.pallas_call(
        _edge_body,
        grid=grid,
        in_specs=[
            pl.BlockSpec((be, d), row),
            pl.BlockSpec((be, d), row),
            pl.BlockSpec((be, d), row),
            pl.BlockSpec((be, dsh), row),
            pl.BlockSpec((be, eb), row),
            pl.BlockSpec((be, 1), row),
            pl.BlockSpec((3 * d, hid), full),
            pl.BlockSpec((1, hid), full),
            pl.BlockSpec((hid, dsh), full),
            pl.BlockSpec((eb, fch), full),
            pl.BlockSpec((fch, 3 * d), full),
            pl.BlockSpec((3 * d, hid), full),
            pl.BlockSpec((1, hid), full),
            pl.BlockSpec((hid, d), full),
        ],
        out_specs=[
            pl.BlockSpec((be, d), row),
            pl.BlockSpec((be, d), row),
        ],
        out_shape=[
            jax.ShapeDtypeStruct((e, d), jnp.float32),
            jax.ShapeDtypeStruct((e, d), jnp.float32),
        ],
        compiler_params=pltpu.CompilerParams(
            dimension_semantics=("parallel",)),
    )(he, hs, hd, fe, fes, normc, we1, be1, we2, wf1, wf2, wu1, bu1, wu2)


def _node_body(hn_ref, p0_ref, p1_ref, wn1_ref, bn1_ref, wn2_ref, out_ref):
    f32 = jnp.float32
    hn = hn_ref[...]
    d = hn.shape[1]
    nf = p0_ref[...] + p1_ref[...]
    a = jnp.dot(hn.astype(jnp.bfloat16), wn1_ref[0:d, :],
                preferred_element_type=f32)
    a += jnp.dot(nf.astype(jnp.bfloat16), wn1_ref[d:2 * d, :],
                 preferred_element_type=f32)
    a += bn1_ref[...]
    h = a * (0.5 * jnp.tanh(0.5 * a) + 0.5)
    out_ref[...] = hn + jnp.dot(h.astype(jnp.bfloat16), wn2_ref[...],
                                preferred_element_type=f32)


def _node_call(hn, p0, p1, wn1, bn1, wn2, bn):
    n, d = hn.shape
    hid = wn1.shape[1]
    grid = (n // bn,)
    row = lambda i: (i, 0)
    full = lambda i: (0, 0)
    return pl.pallas_call(
        _node_body,
        grid=grid,
        in_specs=[
            pl.BlockSpec((bn, d), row),
            pl.BlockSpec((bn, d), row),
            pl.BlockSpec((bn, d), row),
            pl.BlockSpec((2 * d, hid), full),
            pl.BlockSpec((1, hid), full),
            pl.BlockSpec((hid, d), full),
        ],
        out_specs=pl.BlockSpec((bn, d), row),
        out_shape=jax.ShapeDtypeStruct((n, d), jnp.float32),
        compiler_params=pltpu.CompilerParams(
            dimension_semantics=("parallel",)),
    )(hn, p0, p1, wn1, bn1, wn2)


def kernel(hn, he, fe, fes, norm, edge_index, We1, be1, We2, Wf1, Wf2,
           Wu1, bu1, Wu2, Wn1, bn1, Wn2):
    n, d = hn.shape
    e = he.shape[0]
    bf = jnp.bfloat16
    src = edge_index[0]
    dst = edge_index[1]

    hs, hd = _gather_call(hn, src, dst)

    he_new, scaled = _edge_call(
        he, hs, hd, fe, fes, norm.reshape(e, 1),
        We1.astype(bf), be1.reshape(1, -1), We2, Wf1, Wf2,
        Wu1.astype(bf), bu1.reshape(1, -1), Wu2.astype(bf), be=1280)

    np_ = ((n + NS * 8 - 1) // (NS * 8)) * (NS * 8)
    partials = _scatter_call(scaled, dst, jnp.zeros((np_, d), jnp.float32))

    hn_new = _node_call(hn, partials[0, :n], partials[1, :n],
                        Wn1.astype(bf), bn1.reshape(1, -1), Wn2.astype(bf),
                        bn=2000)
    return hn_new, he_new


# speedup vs baseline: 2.0808x; 1.0111x over previous
import functools
import math

import jax
import jax.numpy as jnp
import numpy as np
from jax import lax
from jax.experimental import pallas as pl
from jax.experimental.pallas import tpu as pltpu
from jax.experimental.pallas import tpu_sc as plsc

NC = 2
NS = 16
NW = NC * NS
CH = 80

def _mk_mesh():
    return plsc.VectorSubcoreMesh(core_axis_name="c", subcore_axis_name="s",
                                  num_cores=NC, num_subcores=NS)


def _gather_call(hn_f32, src, dst):
    n, d = hn_f32.shape
    e = src.shape[0]
    ew = e // NW
    nch = ew // CH

    @functools.partial(
        pl.kernel,
        mesh=_mk_mesh(),
        out_type=(
            jax.ShapeDtypeStruct((e, d), jnp.float32),
            jax.ShapeDtypeStruct((e, d), jnp.float32),
        ),
        scratch_types=[
            pltpu.VMEM((CH,), jnp.int32),
            pltpu.VMEM((CH,), jnp.int32),
            pltpu.VMEM((CH, d), jnp.float32),
            pltpu.VMEM((CH, d), jnp.float32),
            pltpu.SemaphoreType.DMA,
            pltpu.SemaphoreType.DMA,
        ],
    )
    def k(hn_hbm, src_hbm, dst_hbm, hs_hbm, hd_hbm, sidx, didx, srows, drows,
          sem1, sem2):
        wid = lax.axis_index("s") * NC + lax.axis_index("c")

        def body(j, carry):
            base = wid * ew + j * CH
            pltpu.sync_copy(src_hbm.at[pl.ds(base, CH)], sidx)
            pltpu.sync_copy(dst_hbm.at[pl.ds(base, CH)], didx)
            c1 = pltpu.async_copy(hn_hbm.at[sidx], srows, sem1)
            c2 = pltpu.async_copy(hn_hbm.at[didx], drows, sem2)
            c1.wait()
            c2.wait()
            pltpu.sync_copy(srows, hs_hbm.at[pl.ds(base, CH)])
            pltpu.sync_copy(drows, hd_hbm.at[pl.ds(base, CH)])
            return carry

        lax.fori_loop(0, nch, body, 0)

    return k(hn_f32, src, dst)


def _scatter_call(scaled_parts, dst, zrows):
    d = scaled_parts[0].shape[1]
    np_ = zrows.shape[0]
    npt = np_ // NS

    @functools.partial(
        pl.kernel,
        mesh=_mk_mesh(),
        out_type=jax.ShapeDtypeStruct((NC, np_, d), jnp.float32),
        scratch_types=[
            pltpu.VMEM((CH,), jnp.int32),
            pltpu.VMEM((CH, d), jnp.float32),
            pltpu.VMEM_SHARED((np_, d), jnp.float32),
        ],
    )
    def k(*refs):
        nparts = len(scaled_parts)
        part_refs = refs[:nparts]
        dst_hbm, z_hbm, out_hbm, idx_v, rows_v, shared = refs[nparts:]
        cid = lax.axis_index("c")
        sid = lax.axis_index("s")
        wid = sid * NC + cid
        pltpu.sync_copy(z_hbm.at[pl.ds(sid * npt, npt)],
                        shared.at[pl.ds(sid * npt, npt)])
        plsc.subcore_barrier()

        base_p = 0
        for pi in range(nparts):
            s_hbm = part_refs[pi]
            e_p = scaled_parts[pi].shape[0]
            ewp = e_p // NW
            nchp = ewp // CH

            def body(j, carry, s_hbm=s_hbm, base_p=base_p, ewp=ewp):
                off = wid * ewp + j * CH
                pltpu.sync_copy(dst_hbm.at[pl.ds(base_p + off, CH)], idx_v)
                pltpu.sync_copy(s_hbm.at[pl.ds(off, CH)], rows_v)
                pltpu.sync_copy(rows_v, shared.at[idx_v], add=True)
                return carry

            lax.fori_loop(0, nchp, body, 0)
            base_p += e_p
        plsc.subcore_barrier()
        pltpu.sync_copy(shared.at[pl.ds(sid * npt, npt)],
                        out_hbm.at[cid, pl.ds(sid * npt, npt)])

    return k(*scaled_parts, dst, zrows)


_SUBTILES = 5


def _edge_body(he_ref, hs_ref, hd_ref, fe_ref, fes_ref, norm_ref,
               we1_ref, be1_ref, we2_ref, wf1_ref, wf2_ref,
               wu1_ref, bu1_ref, wu2_ref, he_new_ref, scaled_ref):
    f32 = jnp.float32
    bf = jnp.bfloat16
    be, d = he_ref.shape
    eb = fes_ref.shape[1]
    fch = wf1_ref.shape[1]
    st = be // _SUBTILES
    for t in range(_SUBTILES):
        rs = pl.ds(t * st, st)
        he = he_ref[rs, :]
        hs = hs_ref[rs, :].astype(bf)
        hd = hd_ref[rs, :].astype(bf)
        x1 = jnp.dot(he.astype(bf), we1_ref[0:d, :],
                     preferred_element_type=f32)
        x1 += jnp.dot(hs, we1_ref[d:2 * d, :], preferred_element_type=f32)
        x1 += jnp.dot(hd, we1_ref[2 * d:3 * d, :], preferred_element_type=f32)
        x1 += be1_ref[...]
        h1 = x1 * (0.5 * jnp.tanh(0.5 * x1) + 0.5)
        v = jnp.dot(h1, we2_ref[...], preferred_element_type=f32)
        r = jnp.maximum(
            jnp.dot(fes_ref[rs, :], wf1_ref[...],
                    preferred_element_type=f32), 0.0)
        w3 = jnp.dot(r, wf2_ref[...], preferred_element_type=f32) * (
            1.0 / math.sqrt(eb) / math.sqrt(fch))
        p = v * fe_ref[rs, :]
        d0 = jnp.sum(p[:, 0:1], axis=1, keepdims=True)
        d1 = jnp.sum(p[:, 1:4], axis=1, keepdims=True) * (
            1.0 / math.sqrt(3.0))
        d2 = jnp.sum(p[:, 4:9], axis=1, keepdims=True) * (
            1.0 / math.sqrt(5.0))
        tp = (w3[:, 0:d] * d0 + w3[:, d:2 * d] * d1 +
              w3[:, 2 * d:3 * d] * d2) * (1.0 / math.sqrt(3.0))
        u1 = jnp.dot(tp.astype(bf), wu1_ref[0:d, :],
                     preferred_element_type=f32)
        u1 += jnp.dot(hs, wu1_ref[d:2 * d, :], preferred_element_type=f32)
        u1 += jnp.dot(hd, wu1_ref[2 * d:3 * d, :], preferred_element_type=f32)
        u1 += bu1_ref[...]
        h2 = u1 * (0.5 * jnp.tanh(0.5 * u1) + 0.5)
        he_new = he + jnp.dot(h2.astype(bf), wu2_ref[...],
                              preferred_element_type=f32)
        he_new_ref[rs, :] = he_new
        scaled_ref[rs, :] = he_new * norm_ref[rs, :]


def _edge_call(he, hs, hd, fe, fes, normc, we1, be1, we2, wf1, wf2,
               wu1, bu1, wu2, be, base, epart):
    e, d = he.shape
    hid = we1.shape[1]
    dsh = fe.shape[1]
    eb = fes.shape[1]
    fch = wf1.shape[1]
    grid = (epart // be,)
    off = base // be
    rowo = lambda i: (i + off, 0)
    row = lambda i: (i, 0)
    full = lambda i: (0, 0)
    return pl.pallas_call(
        _edge_body,
        grid=grid,
        in_specs=[
            pl.BlockSpec((be, d), rowo),
            pl.BlockSpec((be, d), row),
            pl.BlockSpec((be, d), row),
            pl.BlockSpec((be, dsh), rowo),
            pl.BlockSpec((be, eb), rowo),
            pl.BlockSpec((be, 1), rowo),
            pl.BlockSpec((3 * d, hid), full),
            pl.BlockSpec((1, hid), full),
            pl.BlockSpec((hid, dsh), full),
            pl.BlockSpec((eb, fch), full),
            pl.BlockSpec((fch, 3 * d), full),
            pl.BlockSpec((3 * d, hid), full),
            pl.BlockSpec((1, hid), full),
            pl.BlockSpec((hid, d), full),
        ],
        out_specs=[
            pl.BlockSpec((be, d), row),
            pl.BlockSpec((be, d), row),
        ],
        out_shape=[
            jax.ShapeDtypeStruct((epart, d), jnp.float32),
            jax.ShapeDtypeStruct((epart, d), jnp.float32),
        ],
        compiler_params=pltpu.CompilerParams(
            dimension_semantics=("parallel",)),
    )(he, hs, hd, fe, fes, normc, we1, be1, we2, wf1, wf2, wu1, bu1, wu2)


def _node_body(hn_ref, p0_ref, p1_ref, wn1_ref, bn1_ref, wn2_ref, out_ref):
    f32 = jnp.float32
    hn = hn_ref[...]
    d = hn.shape[1]
    nf = p0_ref[...] + p1_ref[...]
    a = jnp.dot(hn.astype(jnp.bfloat16), wn1_ref[0:d, :],
                preferred_element_type=f32)
    a += jnp.dot(nf.astype(jnp.bfloat16), wn1_ref[d:2 * d, :],
                 preferred_element_type=f32)
    a += bn1_ref[...]
    h = a * (0.5 * jnp.tanh(0.5 * a) + 0.5)
    out_ref[...] = hn + jnp.dot(h.astype(jnp.bfloat16), wn2_ref[...],
                                preferred_element_type=f32)


def _node_call(hn, p0, p1, wn1, bn1, wn2, bn):
    n, d = hn.shape
    hid = wn1.shape[1]
    grid = (n // bn,)
    row = lambda i: (i, 0)
    full = lambda i: (0, 0)
    return pl.pallas_call(
        _node_body,
        grid=grid,
        in_specs=[
            pl.BlockSpec((bn, d), row),
            pl.BlockSpec((bn, d), row),
            pl.BlockSpec((bn, d), row),
            pl.BlockSpec((2 * d, hid), full),
            pl.BlockSpec((1, hid), full),
            pl.BlockSpec((hid, d), full),
        ],
        out_specs=pl.BlockSpec((bn, d), row),
        out_shape=jax.ShapeDtypeStruct((n, d), jnp.float32),
        compiler_params=pltpu.CompilerParams(
            dimension_semantics=("parallel",)),
    )(hn, p0, p1, wn1, bn1, wn2)


def kernel(hn, he, fe, fes, norm, edge_index, We1, be1, We2, Wf1, Wf2,
           Wu1, bu1, Wu2, Wn1, bn1, Wn2):
    n, d = hn.shape
    e = he.shape[0]
    bf = jnp.bfloat16
    src = edge_index[0]
    dst = edge_index[1]

    unit = NW * CH
    be = 1280
    nparts = 2
    per = (e // nparts) // unit * unit
    bounds = [per * i for i in range(nparts)] + [e]
    normc = norm.reshape(e, 1)
    wargs = (We1.astype(bf), be1.reshape(1, -1), We2, Wf1, Wf2,
             Wu1.astype(bf), bu1.reshape(1, -1), Wu2.astype(bf))

    gathered = []
    for p in range(nparts):
        b0, b1 = bounds[p], bounds[p + 1]
        gathered.append(_gather_call(hn, src[b0:b1], dst[b0:b1]))

    he_parts, sc_parts = [], []
    for p in range(nparts):
        b0, b1 = bounds[p], bounds[p + 1]
        hs, hd = gathered[p]
        he_p, sc_p = _edge_call(he, hs, hd, fe, fes, normc, *wargs,
                                be=be, base=b0, epart=b1 - b0)
        he_parts.append(he_p)
        sc_parts.append(sc_p)
    he_new = jnp.concatenate(he_parts, axis=0)

    np_ = ((n + NS * 8 - 1) // (NS * 8)) * (NS * 8)
    partials = _scatter_call(tuple(sc_parts), dst,
                             jnp.zeros((np_, d), jnp.float32))

    hn_new = _node_call(hn, partials[0, :n], partials[1, :n],
                        Wn1.astype(bf), bn1.reshape(1, -1), Wn2.astype(bf),
                        bn=2000)
    return hn_new, he_new


# speedup vs baseline: 2.1568x; 1.0365x over previous
import functools
import math

import jax
import jax.numpy as jnp
import numpy as np
from jax import lax
from jax.experimental import pallas as pl
from jax.experimental.pallas import tpu as pltpu
from jax.experimental.pallas import tpu_sc as plsc

NC = 2
NS = 16
NW = NC * NS
CH = 80

def _mk_mesh():
    return plsc.VectorSubcoreMesh(core_axis_name="c", subcore_axis_name="s",
                                  num_cores=NC, num_subcores=NS)


def _gather_call(hn_f32, src, dst):
    n, d = hn_f32.shape
    e = src.shape[0]
    ew = e // NW
    nch = ew // CH

    @functools.partial(
        pl.kernel,
        mesh=_mk_mesh(),
        out_type=(
            jax.ShapeDtypeStruct((e, d), jnp.float32),
            jax.ShapeDtypeStruct((e, d), jnp.float32),
        ),
        scratch_types=[
            pltpu.VMEM((ew,), jnp.int32),
            pltpu.VMEM((ew,), jnp.int32),
            pltpu.VMEM((CH, d), jnp.float32),
            pltpu.VMEM((CH, d), jnp.float32),
            pltpu.VMEM((CH, d), jnp.float32),
            pltpu.VMEM((CH, d), jnp.float32),
        ] + [pltpu.SemaphoreType.DMA] * 8,
    )
    def k(hn_hbm, src_hbm, dst_hbm, hs_hbm, hd_hbm, sidx, didx,
          sr0, sr1, dr0, dr1, ss0, ss1, sd0, sd1, sw0, sw1, sw2, sw3):
        wid = lax.axis_index("s") * NC + lax.axis_index("c")
        base_w = wid * ew
        pltpu.sync_copy(src_hbm.at[pl.ds(base_w, ew)], sidx)
        pltpu.sync_copy(dst_hbm.at[pl.ds(base_w, ew)], didx)

        def pair(jj, carry):
            j0 = jj * 2
            o0 = j0 * CH
            o1 = o0 + CH
            g0 = pltpu.async_copy(hn_hbm.at[sidx.at[pl.ds(o0, CH)]], sr0, ss0)
            g1 = pltpu.async_copy(hn_hbm.at[sidx.at[pl.ds(o1, CH)]], sr1, ss1)
            g2 = pltpu.async_copy(hn_hbm.at[didx.at[pl.ds(o0, CH)]], dr0, sd0)
            g3 = pltpu.async_copy(hn_hbm.at[didx.at[pl.ds(o1, CH)]], dr1, sd1)
            g0.wait()
            w0 = pltpu.async_copy(sr0, hs_hbm.at[pl.ds(base_w + o0, CH)], sw0)
            g1.wait()
            w1 = pltpu.async_copy(sr1, hs_hbm.at[pl.ds(base_w + o1, CH)], sw1)
            g2.wait()
            w2 = pltpu.async_copy(dr0, hd_hbm.at[pl.ds(base_w + o0, CH)], sw2)
            g3.wait()
            w3 = pltpu.async_copy(dr1, hd_hbm.at[pl.ds(base_w + o1, CH)], sw3)
            w0.wait()
            w1.wait()
            w2.wait()
            w3.wait()
            return carry

        lax.fori_loop(0, nch // 2, pair, 0)
        if nch % 2:
            o0 = (nch - 1) * CH
            g0 = pltpu.async_copy(hn_hbm.at[sidx.at[pl.ds(o0, CH)]], sr0, ss0)
            g2 = pltpu.async_copy(hn_hbm.at[didx.at[pl.ds(o0, CH)]], dr0, sd0)
            g0.wait()
            pltpu.sync_copy(sr0, hs_hbm.at[pl.ds(base_w + o0, CH)])
            g2.wait()
            pltpu.sync_copy(dr0, hd_hbm.at[pl.ds(base_w + o0, CH)])

    return k(hn_f32, src, dst)


def _scatter_call(scaled_parts, dst, zrows):
    d = scaled_parts[0].shape[1]
    np_ = zrows.shape[0]
    npt = np_ // NS

    @functools.partial(
        pl.kernel,
        mesh=_mk_mesh(),
        out_type=jax.ShapeDtypeStruct((NC, np_, d), jnp.float32),
        scratch_types=[
            pltpu.VMEM((CH,), jnp.int32),
            pltpu.VMEM((CH,), jnp.int32),
            pltpu.VMEM((CH, d), jnp.float32),
            pltpu.VMEM((CH, d), jnp.float32),
            pltpu.VMEM_SHARED((np_, d), jnp.float32),
        ] + [pltpu.SemaphoreType.DMA] * 6,
    )
    def k(*refs):
        nparts = len(scaled_parts)
        part_refs = refs[:nparts]
        (dst_hbm, z_hbm, out_hbm, ix0, ix1, rb0, rb1, shared,
         si0, si1, sr0, sr1, sa0, sa1) = refs[nparts:]
        cid = lax.axis_index("c")
        sid = lax.axis_index("s")
        wid = sid * NC + cid
        pltpu.sync_copy(z_hbm.at[pl.ds(sid * npt, npt)],
                        shared.at[pl.ds(sid * npt, npt)])
        plsc.subcore_barrier()

        base_p = 0
        for pi in range(nparts):
            s_hbm = part_refs[pi]
            e_p = scaled_parts[pi].shape[0]
            ewp = e_p // NW
            nchp = ewp // CH

            def pair(jj, carry, s_hbm=s_hbm, base_p=base_p, ewp=ewp):
                off0 = wid * ewp + jj * 2 * CH
                off1 = off0 + CH
                i0 = pltpu.async_copy(dst_hbm.at[pl.ds(base_p + off0, CH)],
                                      ix0, si0)
                i1 = pltpu.async_copy(dst_hbm.at[pl.ds(base_p + off1, CH)],
                                      ix1, si1)
                r0 = pltpu.async_copy(s_hbm.at[pl.ds(off0, CH)], rb0, sr0)
                r1 = pltpu.async_copy(s_hbm.at[pl.ds(off1, CH)], rb1, sr1)
                i0.wait()
                r0.wait()
                a0 = pltpu.async_copy(rb0, shared.at[ix0], sa0, add=True)
                i1.wait()
                r1.wait()
                a0.wait()
                a1 = pltpu.async_copy(rb1, shared.at[ix1], sa1, add=True)
                a1.wait()
                return carry

            lax.fori_loop(0, nchp // 2, pair, 0)
            if nchp % 2:
                off0 = wid * ewp + (nchp - 1) * CH
                pltpu.sync_copy(dst_hbm.at[pl.ds(base_p + off0, CH)], ix0)
                pltpu.sync_copy(s_hbm.at[pl.ds(off0, CH)], rb0)
                pltpu.sync_copy(rb0, shared.at[ix0], add=True)
            base_p += e_p
        plsc.subcore_barrier()
        pltpu.sync_copy(shared.at[pl.ds(sid * npt, npt)],
                        out_hbm.at[cid, pl.ds(sid * npt, npt)])

    return k(*scaled_parts, dst, zrows)


_SUBTILES = 5


def _edge_body(he_ref, hs_ref, hd_ref, fe_ref, fes_ref, norm_ref,
               we1_ref, be1_ref, we2_ref, wf1_ref, wf2_ref,
               wu1_ref, bu1_ref, wu2_ref, he_new_ref, scaled_ref):
    f32 = jnp.float32
    bf = jnp.bfloat16
    be, d = he_ref.shape
    eb = fes_ref.shape[1]
    fch = wf1_ref.shape[1]
    st = be // _SUBTILES
    for t in range(_SUBTILES):
        rs = pl.ds(t * st, st)
        he = he_ref[rs, :]
        hs = hs_ref[rs, :].astype(bf)
        hd = hd_ref[rs, :].astype(bf)
        x1 = jnp.dot(he.astype(bf), we1_ref[0:d, :],
                     preferred_element_type=f32)
        x1 += jnp.dot(hs, we1_ref[d:2 * d, :], preferred_element_type=f32)
        x1 += jnp.dot(hd, we1_ref[2 * d:3 * d, :], preferred_element_type=f32)
        x1 += be1_ref[...]
        h1 = x1 * (0.5 * jnp.tanh(0.5 * x1) + 0.5)
        v = jnp.dot(h1, we2_ref[...], preferred_element_type=f32)
        r = jnp.maximum(
            jnp.dot(fes_ref[rs, :], wf1_ref[...],
                    preferred_element_type=f32), 0.0)
        w3 = jnp.dot(r, wf2_ref[...], preferred_element_type=f32) * (
            1.0 / math.sqrt(eb) / math.sqrt(fch))
        p = v * fe_ref[rs, :]
        d0 = jnp.sum(p[:, 0:1], axis=1, keepdims=True)
        d1 = jnp.sum(p[:, 1:4], axis=1, keepdims=True) * (
            1.0 / math.sqrt(3.0))
        d2 = jnp.sum(p[:, 4:9], axis=1, keepdims=True) * (
            1.0 / math.sqrt(5.0))
        tp = (w3[:, 0:d] * d0 + w3[:, d:2 * d] * d1 +
              w3[:, 2 * d:3 * d] * d2) * (1.0 / math.sqrt(3.0))
        u1 = jnp.dot(tp.astype(bf), wu1_ref[0:d, :],
                     preferred_element_type=f32)
        u1 += jnp.dot(hs, wu1_ref[d:2 * d, :], preferred_element_type=f32)
        u1 += jnp.dot(hd, wu1_ref[2 * d:3 * d, :], preferred_element_type=f32)
        u1 += bu1_ref[...]
        h2 = u1 * (0.5 * jnp.tanh(0.5 * u1) + 0.5)
        he_new = he + jnp.dot(h2.astype(bf), wu2_ref[...],
                              preferred_element_type=f32)
        he_new_ref[rs, :] = he_new
        scaled_ref[rs, :] = he_new * norm_ref[rs, :]


def _edge_call(he, hs, hd, fe, fes, normc, we1, be1, we2, wf1, wf2,
               wu1, bu1, wu2, be, base, epart):
    e, d = he.shape
    hid = we1.shape[1]
    dsh = fe.shape[1]
    eb = fes.shape[1]
    fch = wf1.shape[1]
    grid = (epart // be,)
    off = base // be
    rowo = lambda i: (i + off, 0)
    row = lambda i: (i, 0)
    full = lambda i: (0, 0)
    return pl.pallas_call(
        _edge_body,
        grid=grid,
        in_specs=[
            pl.BlockSpec((be, d), rowo),
            pl.BlockSpec((be, d), row),
            pl.BlockSpec((be, d), row),
            pl.BlockSpec((be, dsh), rowo),
            pl.BlockSpec((be, eb), rowo),
            pl.BlockSpec((be, 1), rowo),
            pl.BlockSpec((3 * d, hid), full),
            pl.BlockSpec((1, hid), full),
            pl.BlockSpec((hid, dsh), full),
            pl.BlockSpec((eb, fch), full),
            pl.BlockSpec((fch, 3 * d), full),
            pl.BlockSpec((3 * d, hid), full),
            pl.BlockSpec((1, hid), full),
            pl.BlockSpec((hid, d), full),
        ],
        out_specs=[
            pl.BlockSpec((be, d), row),
            pl.BlockSpec((be, d), row),
        ],
        out_shape=[
            jax.ShapeDtypeStruct((epart, d), jnp.float32),
            jax.ShapeDtypeStruct((epart, d), jnp.float32),
        ],
        compiler_params=pltpu.CompilerParams(
            dimension_semantics=("parallel",)),
    )(he, hs, hd, fe, fes, normc, we1, be1, we2, wf1, wf2, wu1, bu1, wu2)


def _node_body(*refs):
    f32 = jnp.float32
    hn_ref = refs[0]
    nacc = len(refs) - 5
    acc_refs = refs[1:1 + nacc]
    wn1_ref, bn1_ref, wn2_ref, out_ref = refs[1 + nacc:]
    hn = hn_ref[...]
    d = hn.shape[1]
    nf = acc_refs[0][0]
    for r in acc_refs[1:]:
        nf = nf + r[0]
    a = jnp.dot(hn.astype(jnp.bfloat16), wn1_ref[0:d, :],
                preferred_element_type=f32)
    a += jnp.dot(nf.astype(jnp.bfloat16), wn1_ref[d:2 * d, :],
                 preferred_element_type=f32)
    a += bn1_ref[...]
    h = a * (0.5 * jnp.tanh(0.5 * a) + 0.5)
    out_ref[...] = hn + jnp.dot(h.astype(jnp.bfloat16), wn2_ref[...],
                                preferred_element_type=f32)


def _node_call(hn, partials, wn1, bn1, wn2, bn):
    n, d = hn.shape
    hid = wn1.shape[1]
    grid = (n // bn,)
    row = lambda i: (i, 0)
    full = lambda i: (0, 0)
    acc_specs = []
    acc_args = []
    for p in partials:
        for c in range(NC):
            acc_specs.append(
                pl.BlockSpec((1, bn, d), lambda i, c=c: (c, i, 0)))
            acc_args.append(p)
    return pl.pallas_call(
        _node_body,
        grid=grid,
        in_specs=[pl.BlockSpec((bn, d), row)] + acc_specs + [
            pl.BlockSpec((2 * d, hid), full),
            pl.BlockSpec((1, hid), full),
            pl.BlockSpec((hid, d), full),
        ],
        out_specs=pl.BlockSpec((bn, d), row),
        out_shape=jax.ShapeDtypeStruct((n, d), jnp.float32),
        compiler_params=pltpu.CompilerParams(
            dimension_semantics=("parallel",)),
    )(hn, *acc_args, wn1, bn1, wn2)


def kernel(hn, he, fe, fes, norm, edge_index, We1, be1, We2, Wf1, Wf2,
           Wu1, bu1, Wu2, Wn1, bn1, Wn2):
    n, d = hn.shape
    e = he.shape[0]
    bf = jnp.bfloat16
    src = edge_index[0]
    dst = edge_index[1]

    unit = NW * CH
    be = 1280
    nparts = 4
    per = (e // nparts) // unit * unit
    bounds = [per * i for i in range(nparts)] + [e]
    normc = norm.reshape(e, 1)
    wargs = (We1.astype(bf), be1.reshape(1, -1), We2, Wf1, Wf2,
             Wu1.astype(bf), bu1.reshape(1, -1), Wu2.astype(bf))

    gathered = []
    for p in range(nparts):
        b0, b1 = bounds[p], bounds[p + 1]
        gathered.append(_gather_call(hn, src[b0:b1], dst[b0:b1]))

    he_parts, sc_parts = [], []
    for p in range(nparts):
        b0, b1 = bounds[p], bounds[p + 1]
        hs, hd = gathered[p]
        he_p, sc_p = _edge_call(he, hs, hd, fe, fes, normc, *wargs,
                                be=be, base=b0, epart=b1 - b0)
        he_parts.append(he_p)
        sc_parts.append(sc_p)
    he_new = jnp.concatenate(he_parts, axis=0)

    np_ = ((n + NS * 8 - 1) // (NS * 8)) * (NS * 8)
    zrows = jnp.zeros((np_, d), jnp.float32)
    half = nparts // 2
    pa = _scatter_call(tuple(sc_parts[:half]), dst, zrows)
    pb = _scatter_call(tuple(sc_parts[half:]), dst[bounds[half]:], zrows)

    hn_new = _node_call(hn, [pa, pb],
                        Wn1.astype(bf), bn1.reshape(1, -1), Wn2.astype(bf),
                        bn=2000)
    return hn_new, he_new
